# baseline (speedup 1.0000x reference)
import math

import jax
import jax.numpy as jnp
from jax import lax
from jax.experimental import pallas as pl
from jax.experimental.pallas import tpu as pltpu
from jax.experimental.pallas import tpu_sc as plsc

EMBED_DIM = 64
SEQ_LEN = 200
BATCH = 1024
ROWS = BATCH * SEQ_LEN
NC, NS, LANES = 2, 16, 16
NW = NC * NS
RPW = ROWS // NW
CHUNK = 128
NCHUNK = RPW // CHUNK
SCALE = math.sqrt(EMBED_DIM)


def _positional_table():
    pe_len = SEQ_LEN * 2
    pos = jnp.arange(pe_len, dtype=jnp.float32)[:, None]
    i = jnp.arange(0, EMBED_DIM, 2, dtype=jnp.float32)[None, :]
    sin_part = jnp.sin(pos / jnp.power(10000.0, 2.0 * i / EMBED_DIM))
    cos_part = jnp.cos(pos / jnp.power(10000.0, 2.0 * (i + 1.0) / EMBED_DIM))
    pe = jnp.zeros((pe_len, EMBED_DIM), dtype=jnp.float32)
    pe = pe.at[:, 0::2].set(sin_part)
    pe = pe.at[:, 1::2].set(cos_part)
    return pe[:SEQ_LEN]


def _sc_body(idx_hbm, pe_hbm, table_hbm, out_hbm, idx_v, pe_v, buf, sem):
    cid = lax.axis_index("c")
    sid = lax.axis_index("s")
    wid = sid * NC + cid

    pltpu.sync_copy(idx_hbm.at[wid], idx_v)
    pltpu.sync_copy(pe_hbm, pe_v)

    def chunk(j, carry):
        pltpu.async_copy(table_hbm.at[idx_v.at[j]], buf, sem).wait()

        def row(i, c2):
            pos = (j * CHUNK + i) % SEQ_LEN
            for k in range(EMBED_DIM // LANES):
                sl = pl.ds(k * LANES, LANES)
                buf[i, sl] = buf[i, sl] * SCALE + pe_v[pos, sl]
            return c2

        lax.fori_loop(0, CHUNK, row, 0)
        pltpu.sync_copy(buf, out_hbm.at[pl.ds(wid * RPW + j * CHUNK, CHUNK)])
        return carry

    lax.fori_loop(0, NCHUNK, chunk, 0)


_emb = pl.kernel(
    _sc_body,
    out_type=jax.ShapeDtypeStruct((ROWS, EMBED_DIM), jnp.float32),
    mesh=plsc.VectorSubcoreMesh(
        core_axis_name="c", subcore_axis_name="s",
        num_cores=NC, num_subcores=NS,
    ),
    scratch_types=[
        pltpu.VMEM((NCHUNK, CHUNK), jnp.int32),
        pltpu.VMEM((SEQ_LEN, EMBED_DIM), jnp.float32),
        pltpu.VMEM((CHUNK, EMBED_DIM), jnp.float32),
        pltpu.SemaphoreType.DMA,
    ],
    compiler_params=pltpu.CompilerParams(use_tc_tiling_on_sc=False),
)


def kernel(input_seq, embed_weight):
    idx = input_seq.astype(jnp.int32).reshape(NW, NCHUNK, CHUNK)
    out = _emb(idx, _positional_table(), embed_weight)
    return out.reshape(BATCH, SEQ_LEN, EMBED_DIM)


# speedup vs baseline: 1.1898x; 1.1898x over previous
import math

import jax
import jax.numpy as jnp
from jax import lax
from jax.experimental import pallas as pl
from jax.experimental.pallas import tpu as pltpu
from jax.experimental.pallas import tpu_sc as plsc

EMBED_DIM = 64
SEQ_LEN = 200
BATCH = 1024
ROWS = BATCH * SEQ_LEN
NC, NS, LANES = 2, 16, 16
NW = NC * NS
RPW = ROWS // NW
CHUNK = 128
NCHUNK = RPW // CHUNK
NBUF = 5
PE_EXT = SEQ_LEN + CHUNK
SCALE = math.sqrt(EMBED_DIM)


def _positional_table():
    pe_len = SEQ_LEN * 2
    pos = jnp.arange(pe_len, dtype=jnp.float32)[:, None]
    i = jnp.arange(0, EMBED_DIM, 2, dtype=jnp.float32)[None, :]
    sin_part = jnp.sin(pos / jnp.power(10000.0, 2.0 * i / EMBED_DIM))
    cos_part = jnp.cos(pos / jnp.power(10000.0, 2.0 * (i + 1.0) / EMBED_DIM))
    pe = jnp.zeros((pe_len, EMBED_DIM), dtype=jnp.float32)
    pe = pe.at[:, 0::2].set(sin_part)
    pe = pe.at[:, 1::2].set(cos_part)
    pe = pe[:SEQ_LEN]
    return jnp.concatenate([pe, pe[: PE_EXT - SEQ_LEN]], axis=0)


def _sc_body(idx_hbm, pe_hbm, table_hbm, out_hbm, idx_v, pe_v, *rest):
    bufs = rest[:NBUF]
    sg = rest[NBUF:2 * NBUF]
    so = rest[2 * NBUF:]
    cid = lax.axis_index("c")
    sid = lax.axis_index("s")
    wid = sid * NC + cid
    out_base = wid * RPW

    pltpu.sync_copy(idx_hbm.at[wid], idx_v)
    pltpu.sync_copy(pe_hbm, pe_v)

    for b in range(NBUF - 1):
        pltpu.async_copy(table_hbm.at[idx_v.at[b]], bufs[b], sg[b])

    def group(g, carry):
        for b in range(NBUF):
            j = g * NBUF + b
            buf = bufs[b]
            pltpu.make_async_copy(table_hbm.at[idx_v.at[j]], buf, sg[b]).wait()

            p0 = (j * CHUNK) % SEQ_LEN

            @plsc.parallel_loop(0, CHUNK, step=1, unroll=4)
            def _row(i):
                for k in range(EMBED_DIM // LANES):
                    sl = pl.ds(k * LANES, LANES)
                    buf[i, sl] = buf[i, sl] * SCALE + pe_v[p0 + i, sl]

            pltpu.async_copy(
                buf, out_hbm.at[pl.ds(out_base + j * CHUNK, CHUNK)], so[b])

            bn = (b - 1) % NBUF

            def _retire():
                pltpu.make_async_copy(
                    bufs[bn], out_hbm.at[pl.ds(0, CHUNK)], so[bn]).wait()

            def _refill():
                jn = j + NBUF - 1
                pltpu.async_copy(table_hbm.at[idx_v.at[jn]], bufs[bn], sg[bn])

            if b == 0:
                pl.when(g >= 1)(_retire)
                _refill()
            else:
                _retire()
                pl.when(g * NBUF + b + NBUF - 1 < NCHUNK)(_refill)
        return carry

    lax.fori_loop(0, NCHUNK // NBUF, group, 0)

    bl = (NCHUNK - 1) % NBUF
    pltpu.make_async_copy(bufs[bl], out_hbm.at[pl.ds(0, CHUNK)], so[bl]).wait()


_emb = pl.kernel(
    _sc_body,
    out_type=jax.ShapeDtypeStruct((ROWS, EMBED_DIM), jnp.float32),
    mesh=plsc.VectorSubcoreMesh(
        core_axis_name="c", subcore_axis_name="s",
        num_cores=NC, num_subcores=NS,
    ),
    scratch_types=(
        [pltpu.VMEM((NCHUNK, CHUNK), jnp.int32),
         pltpu.VMEM((PE_EXT, EMBED_DIM), jnp.float32)]
        + [pltpu.VMEM((CHUNK, EMBED_DIM), jnp.float32) for _ in range(NBUF)]
        + [pltpu.SemaphoreType.DMA for _ in range(2 * NBUF)]
    ),
    compiler_params=pltpu.CompilerParams(use_tc_tiling_on_sc=False),
)


def kernel(input_seq, embed_weight):
    idx = input_seq.astype(jnp.int32).reshape(NW, NCHUNK, CHUNK)
    out = _emb(idx, _positional_table(), embed_weight)
    return out.reshape(BATCH, SEQ_LEN, EMBED_DIM)
